# baseline (speedup 1.0000x reference)
import functools

import jax
import jax.numpy as jnp
from jax import lax
from jax.experimental import pallas as pl
from jax.experimental.pallas import tpu as pltpu
from jax.experimental.pallas import tpu_sc as plsc

NC = 2
NS = 16
NW = NC * NS
LANES = 16
SPLIT = 104
NBUF = 2


def _sc_sums(idx_flat, emb_table, batch, seq, dim):
    b_per_w = batch // NW
    rest = seq - SPLIT
    mesh = plsc.VectorSubcoreMesh(core_axis_name="c", subcore_axis_name="s")

    @functools.partial(
        pl.kernel,
        mesh=mesh,
        compiler_params=pltpu.CompilerParams(use_tc_tiling_on_sc=False),
        out_type=jax.ShapeDtypeStruct((batch, dim), jnp.float32),
        scratch_types=[
            pltpu.VMEM((b_per_w, seq), jnp.int32),
            pltpu.VMEM((NBUF, SPLIT, dim), jnp.float32),
            pltpu.VMEM((NBUF, rest, dim), jnp.float32),
            pltpu.VMEM((b_per_w, dim), jnp.float32),
            pltpu.SemaphoreType.DMA,
            pltpu.SemaphoreType.DMA,
        ],
    )
    def sums_kernel(idx_hbm, table_hbm, out_hbm,
                    idx_v, buf_a, buf_b, acc, gs0, gs1):
        gs = [gs0, gs1]
        wid = lax.axis_index("s") * NC + lax.axis_index("c")
        base = wid * b_per_w
        pltpu.sync_copy(idx_hbm.at[pl.ds(base, b_per_w)], idx_v)

        def gather_pair(item, slot):
            pltpu.async_copy(
                table_hbm.at[idx_v.at[item, pl.ds(0, SPLIT)]],
                buf_a.at[slot], gs[slot])
            pltpu.async_copy(
                table_hbm.at[idx_v.at[item, pl.ds(SPLIT, rest)]],
                buf_b.at[slot], gs[slot])

        def gather_wait(item, slot):
            pltpu.make_async_copy(
                table_hbm.at[idx_v.at[item, pl.ds(0, SPLIT)]],
                buf_a.at[slot], gs[slot]).wait()
            pltpu.make_async_copy(
                table_hbm.at[idx_v.at[item, pl.ds(SPLIT, rest)]],
                buf_b.at[slot], gs[slot]).wait()

        def reduce_rows(buf, n, carry):
            def body(r8, c):
                r = pl.multiple_of(r8 * 8, 8)
                s0, s1 = c
                for u in range(8):
                    s0 = s0 + buf[r + u, pl.ds(0, LANES)]
                    s1 = s1 + buf[r + u, pl.ds(LANES, LANES)]
                return (s0, s1)

            return lax.fori_loop(0, n // 8, body, carry)

        for b in range(NBUF):
            gather_pair(b, b)

        @pl.loop(0, b_per_w, step=NBUF)
        def _(g):
            for b in range(NBUF):
                item = g + b
                gather_wait(item, b)
                zero = jnp.zeros((LANES,), jnp.float32)
                s0, s1 = reduce_rows(buf_a.at[b], SPLIT, (zero, zero))
                s0, s1 = reduce_rows(buf_b.at[b], rest, (s0, s1))
                acc[item, pl.ds(0, LANES)] = s0
                acc[item, pl.ds(LANES, LANES)] = s1

                @pl.when(item + NBUF < b_per_w)
                def _():
                    gather_pair(item + NBUF, b)

        pltpu.sync_copy(acc, out_hbm.at[pl.ds(wid * b_per_w, b_per_w)])

    return sums_kernel(idx_flat, emb_table)


def _tc_head(sums, w_out, b_out, seq):
    batch, dim = sums.shape

    def head_kernel(s_ref, w_ref, b_ref, o_ref):
        bow = s_ref[...] * (1.0 / seq)
        w = w_ref[...]
        t0 = jnp.sum(bow * w[0:1, :], axis=1, keepdims=True) + b_ref[0, 0]
        t1 = jnp.sum(bow * w[1:2, :], axis=1, keepdims=True) + b_ref[0, 1]
        m = jnp.maximum(t0, t1)
        lse = m + jnp.log(jnp.exp(t0 - m) + jnp.exp(t1 - m))
        o_ref[...] = jnp.concatenate([t0 - lse, t1 - lse], axis=1)

    return pl.pallas_call(
        head_kernel,
        out_shape=jax.ShapeDtypeStruct((batch, 2), jnp.float32),
        in_specs=[
            pl.BlockSpec(memory_space=pltpu.VMEM),
            pl.BlockSpec(memory_space=pltpu.VMEM),
            pl.BlockSpec(memory_space=pltpu.SMEM),
        ],
        out_specs=pl.BlockSpec(memory_space=pltpu.VMEM),
    )(sums, w_out, b_out.reshape(1, 2))


def kernel(input, emb_table, W_out, b_out):
    batch, seq = input.shape
    dim = emb_table.shape[1]
    idx = input.astype(jnp.int32)
    sums = _sc_sums(idx, emb_table, batch, seq, dim)
    return _tc_head(sums, W_out, b_out, seq)


# speedup vs baseline: 1.0600x; 1.0600x over previous
import functools

import jax
import jax.numpy as jnp
from jax import lax
from jax.experimental import pallas as pl
from jax.experimental.pallas import tpu as pltpu
from jax.experimental.pallas import tpu_sc as plsc

NC = 2
NS = 16
NW = NC * NS
LANES = 16
SPLIT = 104
NBUF = 4


def _sc_sums(idx_flat, emb_table, batch, seq, dim):
    b_per_w = batch // NW
    rest = seq - SPLIT
    mesh = plsc.VectorSubcoreMesh(core_axis_name="c", subcore_axis_name="s")

    @functools.partial(
        pl.kernel,
        mesh=mesh,
        compiler_params=pltpu.CompilerParams(use_tc_tiling_on_sc=False),
        out_type=jax.ShapeDtypeStruct((batch, dim), jnp.float32),
        scratch_types=[
            pltpu.VMEM((b_per_w, seq), jnp.int32),
            pltpu.VMEM((NBUF, SPLIT, dim), jnp.float32),
            pltpu.VMEM((NBUF, rest, dim), jnp.float32),
            pltpu.VMEM((b_per_w, dim), jnp.float32),
            pltpu.SemaphoreType.DMA,
            pltpu.SemaphoreType.DMA,
            pltpu.SemaphoreType.DMA,
            pltpu.SemaphoreType.DMA,
        ],
    )
    def sums_kernel(idx_hbm, table_hbm, out_hbm,
                    idx_v, buf_a, buf_b, acc, gs0, gs1, gs2, gs3):
        gs = [gs0, gs1, gs2, gs3]
        wid = lax.axis_index("s") * NC + lax.axis_index("c")
        base = wid * b_per_w
        pltpu.sync_copy(idx_hbm.at[pl.ds(base, b_per_w)], idx_v)

        def gather_pair(item, slot):
            pltpu.async_copy(
                table_hbm.at[idx_v.at[item, pl.ds(0, SPLIT)]],
                buf_a.at[slot], gs[slot])
            pltpu.async_copy(
                table_hbm.at[idx_v.at[item, pl.ds(SPLIT, rest)]],
                buf_b.at[slot], gs[slot])

        def gather_wait(item, slot):
            pltpu.make_async_copy(
                table_hbm.at[idx_v.at[item, pl.ds(0, SPLIT)]],
                buf_a.at[slot], gs[slot]).wait()
            pltpu.make_async_copy(
                table_hbm.at[idx_v.at[item, pl.ds(SPLIT, rest)]],
                buf_b.at[slot], gs[slot]).wait()

        def reduce_rows(buf, n, carry):
            def body(r8, c):
                r = pl.multiple_of(r8 * 8, 8)
                s0, s1 = c
                for u in range(8):
                    s0 = s0 + buf[r + u, pl.ds(0, LANES)]
                    s1 = s1 + buf[r + u, pl.ds(LANES, LANES)]
                return (s0, s1)

            return lax.fori_loop(0, n // 8, body, carry)

        for b in range(NBUF):
            gather_pair(b, b)

        @pl.loop(0, b_per_w, step=NBUF)
        def _(g):
            for b in range(NBUF):
                item = g + b
                gather_wait(item, b)
                zero = jnp.zeros((LANES,), jnp.float32)
                s0, s1 = reduce_rows(buf_a.at[b], SPLIT, (zero, zero))
                s0, s1 = reduce_rows(buf_b.at[b], rest, (s0, s1))
                acc[item, pl.ds(0, LANES)] = s0
                acc[item, pl.ds(LANES, LANES)] = s1

                @pl.when(item + NBUF < b_per_w)
                def _():
                    gather_pair(item + NBUF, b)

        pltpu.sync_copy(acc, out_hbm.at[pl.ds(wid * b_per_w, b_per_w)])

    return sums_kernel(idx_flat, emb_table)


def _tc_head(sums, w_out, b_out, seq):
    batch, dim = sums.shape

    def head_kernel(s_ref, w_ref, b_ref, o_ref):
        bow = s_ref[...] * (1.0 / seq)
        w = w_ref[...]
        t0 = jnp.sum(bow * w[0:1, :], axis=1, keepdims=True) + b_ref[0, 0]
        t1 = jnp.sum(bow * w[1:2, :], axis=1, keepdims=True) + b_ref[0, 1]
        m = jnp.maximum(t0, t1)
        lse = m + jnp.log(jnp.exp(t0 - m) + jnp.exp(t1 - m))
        o_ref[...] = jnp.concatenate([t0 - lse, t1 - lse], axis=1)

    return pl.pallas_call(
        head_kernel,
        out_shape=jax.ShapeDtypeStruct((batch, 2), jnp.float32),
        in_specs=[
            pl.BlockSpec(memory_space=pltpu.VMEM),
            pl.BlockSpec(memory_space=pltpu.VMEM),
            pl.BlockSpec(memory_space=pltpu.SMEM),
        ],
        out_specs=pl.BlockSpec(memory_space=pltpu.VMEM),
    )(sums, w_out, b_out.reshape(1, 2))


def kernel(input, emb_table, W_out, b_out):
    batch, seq = input.shape
    dim = emb_table.shape[1]
    idx = input.astype(jnp.int32)
    sums = _sc_sums(idx, emb_table, batch, seq, dim)
    return _tc_head(sums, W_out, b_out, seq)
